# baseline (speedup 1.0000x reference)
import functools

import jax
import jax.numpy as jnp
import numpy as np
from jax import lax
from jax.experimental import pallas as pl
from jax.experimental.pallas import tpu as pltpu
from jax.experimental.pallas import tpu_sc as plsc

NUM_FEATURES = 4
PER_DIM = 32
NC = 2
NS = 16
NW = NC * NS
L = 16

_UNPACK_PERM = np.concatenate([np.arange(0, 32, 2), np.arange(1, 32, 2)])
_W_PERM = np.concatenate(
    [f * PER_DIM + _UNPACK_PERM for f in range(NUM_FEATURES)]
)


@functools.cache
def _make_gather(batch: int, vocabs: tuple) -> object:
    bpw = batch // NW
    mesh = plsc.VectorSubcoreMesh(core_axis_name="c", subcore_axis_name="s")
    out_type = jax.ShapeDtypeStruct((NUM_FEATURES * PER_DIM, batch), jnp.float32)
    scratch = (
        [pltpu.VMEM((bpw,), jnp.int32) for _ in range(NUM_FEATURES)]
        + [pltpu.VMEM((bpw, PER_DIM), jnp.bfloat16) for _ in range(NUM_FEATURES)]
        + [pltpu.VMEM((bpw, PER_DIM), jnp.float32)]
        + [pltpu.VMEM((PER_DIM, bpw), jnp.float32) for _ in range(2)]
        + [pltpu.SemaphoreType.DMA, pltpu.SemaphoreType.DMA]
    )

    @functools.partial(
        pl.kernel, mesh=mesh, out_type=out_type, scratch_types=scratch,
        compiler_params=pltpu.CompilerParams(
            use_tc_tiling_on_sc=False, needs_layout_passes=False),
    )
    def gather_kernel(i0, i1, i2, i3, t0, t1, t2, t3, et,
                      x0, x1, x2, x3, r0, r1, r2, r3, rf, c0, c1,
                      sem, osem):
        wid = lax.axis_index("s") * NC + lax.axis_index("c")
        base = wid * bpw
        ids = (i0, i1, i2, i3)
        tabs = (t0, t1, t2, t3)
        idxs = (x0, x1, x2, x3)
        rows = (r0, r1, r2, r3)
        cols = (c0, c1)
        cps = []
        for f in range(NUM_FEATURES):
            pltpu.sync_copy(ids[f].at[pl.ds(base, bpw)], idxs[f])
            cps.append(pltpu.async_copy(tabs[f].at[idxs[f]], rows[f], sem))
        ocps = []
        for f in range(NUM_FEATURES):
            cps[f].wait()
            if f >= 2:
                ocps[f - 2].wait()
            rbuf = rows[f]
            cbuf = cols[f % 2]

            def unpack_body(r, *, _r=rbuf):
                lo, hi = plsc.unpack(
                    _r[r, :], format=plsc.PackFormat.INTERLEAVED
                )
                rf[r, pl.ds(0, L)] = lo
                rf[r, pl.ds(L, L)] = hi

            pl.loop(0, bpw)(unpack_body)

            def col_body(c, *, _c=cbuf):
                for g in range(bpw // L):
                    ridx = lax.iota(jnp.int32, L) + g * L
                    cidx = jnp.zeros((L,), jnp.int32) + c
                    _c[c, pl.ds(g * L, L)] = plsc.load_gather(rf, [ridx, cidx])

            pl.loop(0, PER_DIM)(col_body)
            ocps.append(pltpu.async_copy(
                cbuf,
                et.at[pl.ds(f * PER_DIM, PER_DIM), pl.ds(base, bpw)],
                osem,
            ))
        ocps[-2].wait()
        ocps[-1].wait()

    return gather_kernel


def _proj_body(et, w, b, o):
    acc = lax.dot_general(
        et[...], w[...],
        dimension_numbers=(((0,), (1,)), ((), ())),
        preferred_element_type=jnp.float32,
    )
    o[...] = acc + b[...]


def _project(et, w, b2d):
    total_dim, batch = et.shape
    out_dim = w.shape[0]
    blk = min(batch, 2048)
    grid = (batch // blk,)
    return pl.pallas_call(
        _proj_body,
        grid=grid,
        in_specs=[pl.BlockSpec((total_dim, blk), lambda i: (0, i)),
                  pl.BlockSpec((out_dim, total_dim), lambda i: (0, 0)),
                  pl.BlockSpec((1, out_dim), lambda i: (0, 0))],
        out_specs=pl.BlockSpec((blk, out_dim), lambda i: (i, 0)),
        out_shape=jax.ShapeDtypeStruct((batch, out_dim), jnp.float32),
    )(et, w, b2d)


def kernel(ids0, ids1, ids2, ids3, T0, T1, T2, T3, W, b):
    batch = ids0.shape[0]
    vocabs = tuple(t.shape[0] for t in (T0, T1, T2, T3))
    gather = _make_gather(batch, vocabs)
    et = gather(ids0, ids1, ids2, ids3,
                T0.astype(jnp.bfloat16), T1.astype(jnp.bfloat16),
                T2.astype(jnp.bfloat16), T3.astype(jnp.bfloat16))
    wp = W[:, _W_PERM]
    b2d = b.reshape(1, -1)
    return _project(et, wp, b2d)


# speedup vs baseline: 1.1997x; 1.1997x over previous
import functools

import jax
import jax.numpy as jnp
from jax import lax
from jax.experimental import pallas as pl
from jax.experimental.pallas import tpu as pltpu
from jax.experimental.pallas import tpu_sc as plsc

NUM_FEATURES = 4
PER_DIM = 32
PACK = 128 // PER_DIM
NC = 2
NS = 16
NW = NC * NS
L = 16


@functools.cache
def _make_gather(batch: int, vrows: int) -> object:
    bpw = batch // NW
    half = bpw // 2
    mesh = plsc.VectorSubcoreMesh(core_axis_name="c", subcore_axis_name="s")
    out_type = jax.ShapeDtypeStruct((PER_DIM, batch), jnp.float32)
    scratch = (
        [pltpu.VMEM((bpw,), jnp.int32), pltpu.VMEM((bpw,), jnp.int32)]
        + [pltpu.VMEM((half, 128), jnp.float32) for _ in range(2)]
        + [pltpu.VMEM((PER_DIM, bpw), jnp.float32)]
        + [pltpu.SemaphoreType.DMA, pltpu.SemaphoreType.DMA]
    )

    @functools.partial(
        pl.kernel, mesh=mesh, out_type=out_type, scratch_types=scratch,
        compiler_params=pltpu.CompilerParams(
            use_tc_tiling_on_sc=True, needs_layout_passes=False),
    )
    def gather_kernel(idx_hbm, tq, et, idx_v, idxq_v, r0, r1, cols,
                      sem0, sem1):
        wid = lax.axis_index("s") * NC + lax.axis_index("c")
        base = wid * bpw
        pltpu.sync_copy(idx_hbm.at[pl.ds(base, bpw)], idx_v)

        def qbody(g):
            idxq_v[pl.ds(g * L, L)] = lax.shift_right_logical(
                idx_v[pl.ds(g * L, L)], 2
            )

        pl.loop(0, bpw // L)(qbody)
        rows = (r0, r1)
        sems = (sem0, sem1)
        cps = [
            pltpu.async_copy(
                tq.at[idxq_v.at[pl.ds(h * half, half)]], rows[h], sems[h]
            )
            for h in range(2)
        ]
        for h in range(2):
            cps[h].wait()
            rbuf = rows[h]

            def col_body(c, *, _r=rbuf, _h=h):
                for g in range(half // L):
                    ridx = lax.iota(jnp.int32, L) + g * L
                    idv = idx_v[pl.ds(_h * half + g * L, L)]
                    cidx = lax.shift_left(
                        lax.bitwise_and(idv, jnp.int32(PACK - 1)),
                        jnp.int32(5),
                    ) + c
                    cols[c, pl.ds(_h * half + g * L, L)] = plsc.load_gather(
                        _r, [ridx, cidx]
                    )

            pl.loop(0, PER_DIM)(col_body)
        pltpu.sync_copy(cols, et.at[:, pl.ds(base, bpw)])

    return gather_kernel


def _proj_body(e0, e1, e2, e3, w, b, o):
    acc = b[...]
    for f, e in enumerate((e0, e1, e2, e3)):
        acc += lax.dot_general(
            e[...], w[:, f * PER_DIM:(f + 1) * PER_DIM],
            dimension_numbers=(((0,), (1,)), ((), ())),
            preferred_element_type=jnp.float32,
        )
    o[...] = acc


def _project(ets, w, b2d):
    total_dim = w.shape[1]
    out_dim = w.shape[0]
    batch = ets[0].shape[1]
    blk = min(batch, 2048)
    grid = (batch // blk,)
    e_spec = pl.BlockSpec((PER_DIM, blk), lambda i: (0, i))
    return pl.pallas_call(
        _proj_body,
        grid=grid,
        in_specs=[e_spec, e_spec, e_spec, e_spec,
                  pl.BlockSpec((out_dim, total_dim), lambda i: (0, 0)),
                  pl.BlockSpec((1, out_dim), lambda i: (0, 0))],
        out_specs=pl.BlockSpec((blk, out_dim), lambda i: (i, 0)),
        out_shape=jax.ShapeDtypeStruct((batch, out_dim), jnp.float32),
    )(*ets, w, b2d)


def kernel(ids0, ids1, ids2, ids3, T0, T1, T2, T3, W, b):
    batch = ids0.shape[0]
    ids = (ids0, ids1, ids2, ids3)
    tabs = (T0, T1, T2, T3)
    ets = []
    for f in range(NUM_FEATURES):
        tq = tabs[f].reshape(-1, 128)
        gather = _make_gather(batch, tq.shape[0])
        ets.append(gather(ids[f], tq))
    b2d = b.reshape(1, -1)
    return _project(ets, W, b2d)


# speedup vs baseline: 2.8980x; 2.4157x over previous
import functools

import jax
import jax.numpy as jnp
from jax import lax
from jax.experimental import pallas as pl
from jax.experimental.pallas import tpu as pltpu
from jax.experimental.pallas import tpu_sc as plsc

NUM_FEATURES = 4
PER_DIM = 32
PACK = 128 // PER_DIM
NC = 2
NS = 16
NW = NC * NS
L = 16
NBUF = 4
BIG_VOCAB = 200000


@functools.cache
def _make_block_gather(batch: int, vocab: int) -> object:
    bpw = batch // NW
    vtail = (vocab // 128) * 128
    ntail = vocab - vtail
    qmax = vocab // 128 - 1
    mesh = plsc.VectorSubcoreMesh(core_axis_name="c", subcore_axis_name="s")
    out_type = jax.ShapeDtypeStruct((PER_DIM, batch), jnp.float32)
    scratch = (
        [pltpu.VMEM((bpw + L,), jnp.int32)]
        + [pltpu.VMEM((PER_DIM, 128), jnp.float32) for _ in range(NBUF)]
        + [pltpu.VMEM((ntail * PER_DIM,), jnp.float32)]
        + [pltpu.VMEM((PER_DIM, bpw), jnp.float32)]
        + [pltpu.SemaphoreType.DMA for _ in range(NBUF)]
        + [pltpu.SemaphoreType.DMA]
    )

    @functools.partial(
        pl.kernel, mesh=mesh, out_type=out_type, scratch_types=scratch,
        compiler_params=pltpu.CompilerParams(
            use_tc_tiling_on_sc=True, needs_layout_passes=False),
    )
    def block_gather(idx_hbm, tt, tail_hbm, et,
                     sidx, b0, b1, b2, b3, tailb, cols,
                     s0, s1, s2, s3, tsem):
        wid = lax.axis_index("s") * NC + lax.axis_index("c")
        base = wid * bpw
        rings = (b0, b1, b2, b3)
        sems = (s0, s1, s2, s3)
        pltpu.sync_copy(idx_hbm.at[pl.ds(base, bpw)], sidx.at[pl.ds(0, bpw)])
        pltpu.async_copy(tail_hbm, tailb, tsem).wait()

        def sload(i):
            return sidx[pl.ds(i, L)][0]

        def fire(i, slot):
            q = jnp.minimum(
                lax.shift_right_logical(sload(i), 7), jnp.int32(qmax)
            )
            off = pl.multiple_of(q * 128, 128)
            return pltpu.async_copy(
                tt.at[:, pl.ds(off, 128)], rings[slot], sems[slot]
            )

        for i in range(NBUF):
            fire(i, i)

        def body(i):
            idx = sload(i)
            l = lax.bitwise_and(idx, jnp.int32(127))
            for s in range(NBUF):
                @pl.when(jnp.int32(i % NBUF) == s)
                def _(s=s):
                    pltpu.make_async_copy(
                        tt.at[:, pl.ds(0, 128)], rings[s], sems[s]
                    ).wait()
                    blk = rings[s]
                    tidx = jnp.clip(idx - vtail, 0, ntail - 1) * PER_DIM
                    for h in range(2):
                        cvec = lax.iota(jnp.int32, L) + h * L
                        vals = plsc.load_gather(
                            blk, [cvec, jnp.zeros((L,), jnp.int32) + l]
                        )
                        tvals = plsc.load_gather(tailb, [cvec + tidx])
                        pick = jnp.where(idx >= vtail, tvals, vals)
                        plsc.store_scatter(
                            cols, [cvec, jnp.zeros((L,), jnp.int32) + i], pick
                        )
                    @pl.when(i + NBUF < bpw)
                    def _():
                        fire(i + NBUF, s)

        pl.loop(0, bpw)(body)
        pltpu.sync_copy(cols, et.at[:, pl.ds(base, bpw)])

    return block_gather


@functools.cache
def _make_packed_gather(batch: int, vrows: int) -> object:
    bpw = batch // NW
    half = bpw // 2
    mesh = plsc.VectorSubcoreMesh(core_axis_name="c", subcore_axis_name="s")
    out_type = jax.ShapeDtypeStruct((PER_DIM, batch), jnp.float32)
    scratch = (
        [pltpu.VMEM((bpw,), jnp.int32), pltpu.VMEM((bpw,), jnp.int32)]
        + [pltpu.VMEM((half, 128), jnp.float32) for _ in range(2)]
        + [pltpu.VMEM((PER_DIM, bpw), jnp.float32)]
        + [pltpu.SemaphoreType.DMA, pltpu.SemaphoreType.DMA]
    )

    @functools.partial(
        pl.kernel, mesh=mesh, out_type=out_type, scratch_types=scratch,
        compiler_params=pltpu.CompilerParams(
            use_tc_tiling_on_sc=True, needs_layout_passes=False),
    )
    def packed_gather(idx_hbm, tq, et, idx_v, idxq_v, r0, r1, cols,
                      sem0, sem1):
        wid = lax.axis_index("s") * NC + lax.axis_index("c")
        base = wid * bpw
        pltpu.sync_copy(idx_hbm.at[pl.ds(base, bpw)], idx_v)

        def qbody(g):
            idxq_v[pl.ds(g * L, L)] = lax.shift_right_logical(
                idx_v[pl.ds(g * L, L)], 2
            )

        pl.loop(0, bpw // L)(qbody)
        rows = (r0, r1)
        sems = (sem0, sem1)
        cps = [
            pltpu.async_copy(
                tq.at[idxq_v.at[pl.ds(h * half, half)]], rows[h], sems[h]
            )
            for h in range(2)
        ]
        for h in range(2):
            cps[h].wait()
            rbuf = rows[h]

            def col_body(c, *, _r=rbuf, _h=h):
                for g in range(half // L):
                    ridx = lax.iota(jnp.int32, L) + g * L
                    idv = idx_v[pl.ds(_h * half + g * L, L)]
                    cidx = lax.shift_left(
                        lax.bitwise_and(idv, jnp.int32(PACK - 1)),
                        jnp.int32(5),
                    ) + c
                    cols[c, pl.ds(_h * half + g * L, L)] = plsc.load_gather(
                        _r, [ridx, cidx]
                    )

            pl.loop(0, PER_DIM)(col_body)
        pltpu.sync_copy(cols, et.at[:, pl.ds(base, bpw)])

    return packed_gather


def _proj_body(e0, e1, e2, e3, w, b, o):
    acc = b[...]
    for f, e in enumerate((e0, e1, e2, e3)):
        acc += lax.dot_general(
            e[...], w[:, f * PER_DIM:(f + 1) * PER_DIM],
            dimension_numbers=(((0,), (1,)), ((), ())),
            preferred_element_type=jnp.float32,
        )
    o[...] = acc


def _project(ets, w, b2d):
    total_dim = w.shape[1]
    out_dim = w.shape[0]
    batch = ets[0].shape[1]
    blk = min(batch, 2048)
    grid = (batch // blk,)
    e_spec = pl.BlockSpec((PER_DIM, blk), lambda i: (0, i))
    return pl.pallas_call(
        _proj_body,
        grid=grid,
        in_specs=[e_spec, e_spec, e_spec, e_spec,
                  pl.BlockSpec((out_dim, total_dim), lambda i: (0, 0)),
                  pl.BlockSpec((1, out_dim), lambda i: (0, 0))],
        out_specs=pl.BlockSpec((blk, out_dim), lambda i: (i, 0)),
        out_shape=jax.ShapeDtypeStruct((batch, out_dim), jnp.float32),
    )(*ets, w, b2d)


def kernel(ids0, ids1, ids2, ids3, T0, T1, T2, T3, W, b):
    batch = ids0.shape[0]
    ids = (ids0, ids1, ids2, ids3)
    tabs = (T0, T1, T2, T3)
    ets = []
    for f in range(NUM_FEATURES):
        tab = tabs[f]
        vocab = tab.shape[0]
        if vocab > BIG_VOCAB:
            vtail = (vocab // 128) * 128
            gather = _make_block_gather(batch, vocab)
            ets.append(gather(ids[f], tab.T, tab[vtail:].reshape(-1)))
        else:
            tq = tab.reshape(-1, 128)
            gather = _make_packed_gather(batch, tq.shape[0])
            ets.append(gather(ids[f], tq))
    b2d = b.reshape(1, -1)
    return _project(ets, W, b2d)


# speedup vs baseline: 3.5180x; 1.2139x over previous
import functools

import jax
import jax.numpy as jnp
from jax import lax
from jax.experimental import pallas as pl
from jax.experimental.pallas import tpu as pltpu
from jax.experimental.pallas import tpu_sc as plsc

NUM_FEATURES = 4
PER_DIM = 32
PACK = 128 // PER_DIM
NC = 2
NS = 16
NW = NC * NS
L = 16
NBUF = 8
BIG_VOCAB = 200000


@functools.cache
def _make_block_gather(batch: int, vocab: int) -> object:
    bpw = batch // NW
    vtail = (vocab // 128) * 128
    ntail = vocab - vtail
    qmax = vocab // 128 - 1
    mesh = plsc.VectorSubcoreMesh(core_axis_name="c", subcore_axis_name="s")
    out_type = jax.ShapeDtypeStruct((PER_DIM, batch), jnp.float32)
    scratch = (
        [pltpu.VMEM((bpw + L,), jnp.int32)]
        + [pltpu.VMEM((PER_DIM, 128), jnp.float32) for _ in range(NBUF)]
        + [pltpu.VMEM((ntail * PER_DIM,), jnp.float32)]
        + [pltpu.VMEM((PER_DIM, bpw), jnp.float32)]
        + [pltpu.SemaphoreType.DMA for _ in range(NBUF)]
        + [pltpu.SemaphoreType.DMA]
    )

    @functools.partial(
        pl.kernel, mesh=mesh, out_type=out_type, scratch_types=scratch,
        compiler_params=pltpu.CompilerParams(
            use_tc_tiling_on_sc=True, needs_layout_passes=False),
    )
    def block_gather(idx_hbm, tt, tail_hbm, et, *scr):
        sidx = scr[0]
        rings = scr[1:1 + NBUF]
        tailb = scr[1 + NBUF]
        cols = scr[2 + NBUF]
        sems = scr[3 + NBUF:3 + 2 * NBUF]
        tsem = scr[-1]
        wid = lax.axis_index("s") * NC + lax.axis_index("c")
        base = wid * bpw
        pltpu.sync_copy(idx_hbm.at[pl.ds(base, bpw)], sidx.at[pl.ds(0, bpw)])
        pltpu.async_copy(tail_hbm, tailb, tsem).wait()

        def sload(i):
            return sidx[pl.ds(i, L)][0]

        def fire(i, slot):
            q = jnp.minimum(
                lax.shift_right_logical(sload(i), 7), jnp.int32(qmax)
            )
            off = pl.multiple_of(q * 128, 128)
            return pltpu.async_copy(
                tt.at[:, pl.ds(off, 128)], rings[slot], sems[slot]
            )

        for i in range(NBUF):
            fire(i, i)

        def body(i):
            idx = sload(i)
            l = lax.bitwise_and(idx, jnp.int32(127))
            for s in range(NBUF):
                @pl.when(jnp.int32(i % NBUF) == s)
                def _(s=s):
                    pltpu.make_async_copy(
                        tt.at[:, pl.ds(0, 128)], rings[s], sems[s]
                    ).wait()
                    blk = rings[s]
                    tidx = jnp.clip(idx - vtail, 0, ntail - 1) * PER_DIM
                    for h in range(2):
                        cvec = lax.iota(jnp.int32, L) + h * L
                        vals = plsc.load_gather(
                            blk, [cvec, jnp.zeros((L,), jnp.int32) + l]
                        )
                        tvals = plsc.load_gather(tailb, [cvec + tidx])
                        pick = jnp.where(idx >= vtail, tvals, vals)
                        plsc.store_scatter(
                            cols, [cvec, jnp.zeros((L,), jnp.int32) + i], pick
                        )
                    @pl.when(i + NBUF < bpw)
                    def _():
                        fire(i + NBUF, s)

        pl.loop(0, bpw)(body)
        pltpu.sync_copy(cols, et.at[:, pl.ds(base, bpw)])

    return block_gather


@functools.cache
def _make_packed_gather(batch: int, vrows: int) -> object:
    bpw = batch // NW
    half = bpw // 2
    mesh = plsc.VectorSubcoreMesh(core_axis_name="c", subcore_axis_name="s")
    out_type = jax.ShapeDtypeStruct((PER_DIM, batch), jnp.float32)
    scratch = (
        [pltpu.VMEM((bpw,), jnp.int32), pltpu.VMEM((bpw,), jnp.int32)]
        + [pltpu.VMEM((half, 128), jnp.float32) for _ in range(2)]
        + [pltpu.VMEM((PER_DIM, bpw), jnp.float32)]
        + [pltpu.SemaphoreType.DMA, pltpu.SemaphoreType.DMA]
    )

    @functools.partial(
        pl.kernel, mesh=mesh, out_type=out_type, scratch_types=scratch,
        compiler_params=pltpu.CompilerParams(
            use_tc_tiling_on_sc=True, needs_layout_passes=False),
    )
    def packed_gather(idx_hbm, tq, et, idx_v, idxq_v, r0, r1, cols,
                      sem0, sem1):
        wid = lax.axis_index("s") * NC + lax.axis_index("c")
        base = wid * bpw
        pltpu.sync_copy(idx_hbm.at[pl.ds(base, bpw)], idx_v)

        def qbody(g):
            idxq_v[pl.ds(g * L, L)] = lax.shift_right_logical(
                idx_v[pl.ds(g * L, L)], 2
            )

        pl.loop(0, bpw // L)(qbody)
        rows = (r0, r1)
        sems = (sem0, sem1)
        cps = [
            pltpu.async_copy(
                tq.at[idxq_v.at[pl.ds(h * half, half)]], rows[h], sems[h]
            )
            for h in range(2)
        ]
        for h in range(2):
            cps[h].wait()
            rbuf = rows[h]

            def col_body(c, *, _r=rbuf, _h=h):
                for g in range(half // L):
                    ridx = lax.iota(jnp.int32, L) + g * L
                    idv = idx_v[pl.ds(_h * half + g * L, L)]
                    cidx = lax.shift_left(
                        lax.bitwise_and(idv, jnp.int32(PACK - 1)),
                        jnp.int32(5),
                    ) + c
                    cols[c, pl.ds(_h * half + g * L, L)] = plsc.load_gather(
                        _r, [ridx, cidx]
                    )

            pl.loop(0, PER_DIM)(col_body)
        pltpu.sync_copy(cols, et.at[:, pl.ds(base, bpw)])

    return packed_gather


def _proj_body(e0, e1, e2, e3, w, b, o):
    acc = b[...]
    for f, e in enumerate((e0, e1, e2, e3)):
        acc += lax.dot_general(
            e[...], w[:, f * PER_DIM:(f + 1) * PER_DIM],
            dimension_numbers=(((0,), (1,)), ((), ())),
            preferred_element_type=jnp.float32,
        )
    o[...] = acc


def _project(ets, w, b2d):
    total_dim = w.shape[1]
    out_dim = w.shape[0]
    batch = ets[0].shape[1]
    blk = min(batch, 2048)
    grid = (batch // blk,)
    e_spec = pl.BlockSpec((PER_DIM, blk), lambda i: (0, i))
    return pl.pallas_call(
        _proj_body,
        grid=grid,
        in_specs=[e_spec, e_spec, e_spec, e_spec,
                  pl.BlockSpec((out_dim, total_dim), lambda i: (0, 0)),
                  pl.BlockSpec((1, out_dim), lambda i: (0, 0))],
        out_specs=pl.BlockSpec((blk, out_dim), lambda i: (i, 0)),
        out_shape=jax.ShapeDtypeStruct((batch, out_dim), jnp.float32),
    )(*ets, w, b2d)


def kernel(ids0, ids1, ids2, ids3, T0, T1, T2, T3, W, b):
    batch = ids0.shape[0]
    ids = (ids0, ids1, ids2, ids3)
    tabs = (T0, T1, T2, T3)
    ets = []
    for f in range(NUM_FEATURES):
        tab = tabs[f]
        vocab = tab.shape[0]
        if vocab > BIG_VOCAB:
            vtail = (vocab // 128) * 128
            gather = _make_block_gather(batch, vocab)
            ets.append(gather(ids[f], tab.T, tab[vtail:].reshape(-1)))
        else:
            tq = tab.reshape(-1, 128)
            gather = _make_packed_gather(batch, tq.shape[0])
            ets.append(gather(ids[f], tq))
    b2d = b.reshape(1, -1)
    return _project(ets, W, b2d)
